# baseline (speedup 1.0000x reference)
import functools

import jax
import jax.numpy as jnp
from jax import lax
from jax.experimental import pallas as pl
from jax.experimental.pallas import tpu as pltpu
from jax.experimental.pallas import tpu_sc as plsc

_T = 5
_ALPHA = 0.5
_NC = 2
_NS = 16
_L = 16
_R = 16
_B_SC = 1024
_BB_TC = 1024


def _ln(x):
    b = lax.bitcast_convert_type(x, jnp.int32)
    e = jnp.right_shift(b, 23) & 0xFF
    ef = (e - 127).astype(jnp.float32)
    m = lax.bitcast_convert_type((b & 0x7FFFFF) | 0x3F800000, jnp.float32)
    t = (m - 1.0) / (m + 1.0)
    t2 = t * t
    ln_m = 2.0 * t * (1.0 + t2 * (1.0 / 3.0 + t2 * (1.0 / 5.0 + t2 * (1.0 / 7.0))))
    return ef * 0.6931471805599453 + ln_m


def _make_sc_call(B, D):
    rows_w = _B_SC // (_NC * _NS)
    n_chunks = rows_w // _R
    vecs = D // _L

    mesh = plsc.VectorSubcoreMesh(core_axis_name="c", subcore_axis_name="s")

    @functools.partial(
        pl.kernel,
        mesh=mesh,
        out_type=jax.ShapeDtypeStruct((_T * _B_SC,), jnp.float32),
        scratch_types=(
            [pltpu.VMEM((_R, D), jnp.float32),
             pltpu.VMEM((_R, D), jnp.float32),
             pltpu.VMEM((_R, D), jnp.int32),
             pltpu.VMEM((_R, D), jnp.int32),
             pltpu.VMEM((_T, _R, D), jnp.float32),
             pltpu.VMEM((_T, _R, D), jnp.float32),
             pltpu.VMEM((_T * rows_w,), jnp.float32),
             pltpu.VMEM((_T * rows_w,), jnp.float32),
             pltpu.VMEM((_L,), jnp.float32),
             pltpu.VMEM((_T * _L,), jnp.float32),
             pltpu.SemaphoreType.DMA,
             pltpu.SemaphoreType.DMA]
        ),
    )
    def sc_call(like_hbm, tri_hbm, mask_hbm, cost_hbm, sys_hbm, out_hbm,
                like_a, like_b, tri_a, tri_b, pl_a, pl_b,
                acc_b, out_b, cost_v, sys_v, sem_a, sem_b):
        wid = lax.axis_index("c") * _NS + lax.axis_index("s")
        base = wid * rows_w

        pltpu.sync_copy(cost_hbm, cost_v)
        pltpu.sync_copy(sys_hbm, sys_v)

        like_bufs = (like_a, like_b)
        tri_bufs = (tri_a, tri_b)
        pl_bufs = (pl_a, pl_b)
        sems = (sem_a, sem_b)

        def issue(g):
            i = g % 2
            r0 = base + g * _R
            hs = [pltpu.async_copy(like_hbm.at[pl.ds(r0, _R), :],
                                   like_bufs[i], sems[i]),
                  pltpu.async_copy(tri_hbm.at[pl.ds(r0, _R), :],
                                   tri_bufs[i], sems[i])]
            for t in range(_T):
                hs.append(pltpu.async_copy(mask_hbm.at[t, pl.ds(r0, _R), :],
                                           pl_bufs[i].at[t], sems[i]))
            return hs

        lane = lax.iota(jnp.int32, _L)
        cost_reg = cost_v[...]

        def _shuffle(x, perm):
            return lax.gather(
                x, perm[:, None],
                lax.GatherDimensionNumbers(
                    offset_dims=(), collapsed_slice_dims=(0,),
                    start_index_map=(0,)),
                slice_sizes=(1,),
                mode=lax.GatherScatterMode.PROMISE_IN_BOUNDS)

        def _lanesum(x):
            for k in (1, 2, 4, 8):
                x = x + _shuffle(x, lane ^ k)
            return x

        pending = issue(0)
        for g in range(n_chunks):
            if g + 1 < n_chunks:
                nxt = issue(g + 1)
            else:
                nxt = None
            for h in pending:
                h.wait()
            pending = nxt
            i = g % 2
            like_r, tri_r, pl_r = like_bufs[i], tri_bufs[i], pl_bufs[i]

            def row_body(r, packed):
                def vec_body(v, accs):
                    sl = pl.ds(v * _L, _L)
                    lk = like_r[r, sl]
                    ti = tri_r[r, sl]
                    w = lk * _shuffle(cost_reg, ti)
                    return tuple(accs[t] + w * pl_r[t, r, sl]
                                 for t in range(_T))

                accs = lax.fori_loop(
                    0, vecs, vec_body,
                    tuple(jnp.zeros((_L,), jnp.float32) for _ in range(_T)))
                sel = lane == r
                return tuple(
                    jnp.where(sel, _lanesum(accs[t]), packed[t])
                    for t in range(_T))

            packed = lax.fori_loop(
                0, _R, row_body,
                tuple(jnp.zeros((_L,), jnp.float32) for _ in range(_T)))
            for t in range(_T):
                acc_b[pl.ds(t * rows_w + g * _R, _R)] = packed[t]

        for g in range(rows_w // _L):
            tot = [_ALPHA * sys_v[pl.ds(t * _L, _L)]
                   + (1.0 - _ALPHA) * acc_b[pl.ds(t * rows_w + g * _L, _L)]
                   for t in range(_T)]
            mx = tot[0]
            for t in range(1, _T):
                mx = jnp.maximum(mx, tot[t])
            es = [jnp.exp(tt - mx) for tt in tot]
            s = es[0]
            for t in range(1, _T):
                s = s + es[t]
            lse = _ln(s)
            for t in range(_T):
                out_b[pl.ds(t * rows_w + g * _L, _L)] = tot[t] - mx - lse

        for t in range(_T):
            pltpu.sync_copy(out_b.at[pl.ds(t * rows_w, rows_w)],
                            out_hbm.at[pl.ds(t * _B_SC + base, rows_w)])

    return sc_call


def _tc_body(sc_ref, like_ref, tri_ref, m_ref, out_ref):
    like = like_ref[...]
    tri = tri_ref[...]
    cr = jnp.zeros_like(like)
    for t in range(1, _T):
        cr = jnp.where(tri == t, sc_ref[t], cr)
    w = like * cr

    totals = []
    for t in range(_T):
        cu = jnp.sum(w * m_ref[t], axis=1)
        totals.append(_ALPHA * sc_ref[_T + t] + (1.0 - _ALPHA) * cu)

    mx = totals[0]
    for t in range(1, _T):
        mx = jnp.maximum(mx, totals[t])
    exps = [jnp.exp(tt - mx) for tt in totals]
    s = exps[0]
    for t in range(1, _T):
        s = s + exps[t]
    lse = jnp.log(s)
    for t in range(_T):
        out_ref[t, :] = totals[t] - mx - lse


@functools.partial(jax.jit, static_argnums=())
def kernel(likelihoods, decision_mask, cruelty_parameters, system_parameters,
           disease_triages):
    B, D = likelihoods.shape
    T = decision_mask.shape[2]
    b_tc = B - _B_SC
    off = _B_SC // _BB_TC

    mask_t = jnp.transpose(decision_mask, (2, 0, 1))
    tri = disease_triages.astype(jnp.int32)
    cost = jnp.concatenate([jnp.zeros((1,), jnp.float32),
                            cruelty_parameters.astype(jnp.float32)])
    sysc = jnp.concatenate([jnp.zeros((1,), jnp.float32),
                            system_parameters.astype(jnp.float32)])
    scalars = jnp.concatenate([cost, sysc])
    cost16 = jnp.zeros((_L,), jnp.float32).at[1:T].set(
        cruelty_parameters.astype(jnp.float32))
    sys2 = jnp.broadcast_to(sysc[:, None], (T, _L)).reshape(T * _L)

    out_sc = _make_sc_call(B, D)(likelihoods, tri, mask_t, cost16, sys2)

    out_tc = pl.pallas_call(
        _tc_body,
        grid=(b_tc // _BB_TC,),
        in_specs=[
            pl.BlockSpec(memory_space=pltpu.SMEM),
            pl.BlockSpec((_BB_TC, D), lambda i: (i + off, 0)),
            pl.BlockSpec((_BB_TC, D), lambda i: (i + off, 0)),
            pl.BlockSpec((T, _BB_TC, D), lambda i: (0, i + off, 0)),
        ],
        out_specs=pl.BlockSpec((T, _BB_TC), lambda i: (0, i)),
        out_shape=jax.ShapeDtypeStruct((T, b_tc), jnp.float32),
        compiler_params=pltpu.CompilerParams(
            dimension_semantics=("parallel",)),
    )(scalars, likelihoods, tri, mask_t)

    out = jnp.concatenate([out_sc.reshape(T, _B_SC), out_tc], axis=1)
    return jnp.transpose(out, (1, 0))


# speedup vs baseline: 1.2580x; 1.2580x over previous
import functools

import jax
import jax.numpy as jnp
from jax.experimental import pallas as pl
from jax.experimental.pallas import tpu as pltpu

_T = 5
_ALPHA = 0.5


def _body(sc_ref, like_ref, tri_ref, m_ref, out_ref):
    like = like_ref[...]
    tri = tri_ref[...]
    cr = jnp.zeros_like(like)
    for t in range(1, _T):
        cr = jnp.where(tri == t, sc_ref[t], cr)
    w = like * cr

    totals = []
    for t in range(_T):
        cu = jnp.sum(w * m_ref[t], axis=1)
        totals.append(_ALPHA * sc_ref[_T + t] + (1.0 - _ALPHA) * cu)

    mx = totals[0]
    for t in range(1, _T):
        mx = jnp.maximum(mx, totals[t])
    exps = [jnp.exp(tt - mx) for tt in totals]
    s = exps[0]
    for t in range(1, _T):
        s = s + exps[t]
    lse = jnp.log(s)
    for t in range(_T):
        out_ref[t, :] = totals[t] - mx - lse


@functools.partial(jax.jit, static_argnums=())
def kernel(likelihoods, decision_mask, cruelty_parameters, system_parameters,
           disease_triages):
    B, D = likelihoods.shape
    T = decision_mask.shape[2]
    bB = 1024

    mask_t = jnp.transpose(decision_mask, (2, 0, 1))
    tri = disease_triages.astype(jnp.int32)
    cost = jnp.concatenate([jnp.zeros((1,), jnp.float32),
                            cruelty_parameters.astype(jnp.float32)])
    sysc = jnp.concatenate([jnp.zeros((1,), jnp.float32),
                            system_parameters.astype(jnp.float32)])
    scalars = jnp.concatenate([cost, sysc])

    out = pl.pallas_call(
        _body,
        grid=(B // bB,),
        in_specs=[
            pl.BlockSpec(memory_space=pltpu.SMEM),
            pl.BlockSpec((bB, D), lambda i: (i, 0)),
            pl.BlockSpec((bB, D), lambda i: (i, 0)),
            pl.BlockSpec((T, bB, D), lambda i: (0, i, 0)),
        ],
        out_specs=pl.BlockSpec((T, bB), lambda i: (0, i)),
        out_shape=jax.ShapeDtypeStruct((T, B), jnp.float32),
        compiler_params=pltpu.CompilerParams(
            dimension_semantics=("parallel",)),
    )(scalars, likelihoods, tri, mask_t)
    return jnp.transpose(out, (1, 0))
